# baseline (speedup 1.0000x reference)
import jax
import jax.numpy as jnp
from jax import lax
from jax.experimental import pallas as pl
from jax.experimental.pallas import tpu as pltpu
from jax.experimental.pallas import tpu_sc as plsc

VOCAB = 1000000
EMBED = 64
BATCH = 16384

_info = plsc.get_sparse_core_info()
NC, NS, L = _info.num_cores, _info.num_subcores, _info.num_lanes
NW = NC * NS
BPW = BATCH // NW
CHUNK = 32
NCHUNK = BPW // CHUNK
NVREG = EMBED // 16

TBLK = 512
TGRID = (VOCAB + TBLK - 1) // TBLK


def _tc_transpose_body(t_in, c_in, t_out, c_out):
    t_out[...] = t_in[...].T
    c_out[...] = c_in[...].T


@jax.jit
def _tc_transpose(tt, cc):
    return pl.pallas_call(
        _tc_transpose_body,
        grid=(TGRID,),
        in_specs=[
            pl.BlockSpec((EMBED, TBLK), lambda i: (0, i)),
            pl.BlockSpec((EMBED, TBLK), lambda i: (0, i)),
        ],
        out_specs=[
            pl.BlockSpec((TBLK, EMBED), lambda i: (i, 0)),
            pl.BlockSpec((TBLK, EMBED), lambda i: (i, 0)),
        ],
        out_shape=[
            jax.ShapeDtypeStruct((VOCAB, EMBED), jnp.float32),
            jax.ShapeDtypeStruct((VOCAB, EMBED), jnp.float32),
        ],
    )(tt, cc)


def _sc_body(x_hbm, t_hbm, tgt_hbm, ctx_hbm, out_hbm,
             xidx, tidx, tbuf, cbuf, outv, semt, semc):
    wid = lax.axis_index("s") * NC + lax.axis_index("c")
    base = wid * BPW

    pltpu.sync_copy(x_hbm.at[pl.ds(base, BPW)], xidx)
    pltpu.sync_copy(t_hbm.at[pl.ds(base, BPW)], tidx)

    lane = lax.iota(jnp.int32, L)

    def chunk_step(p, carry):
        cbase = p * CHUNK
        descs = []
        for cc in range(CHUNK // 16):
            tv = tidx[pl.ds(cbase + cc * 16, 16)]
            xv = xidx[pl.ds(cbase + cc * 16, 16)]
            for j in range(16):
                row = cc * 16 + j
                descs.append(pltpu.async_copy(
                    tgt_hbm.at[tv[j]], tbuf.at[row, pl.ds(0, EMBED)], semt))
                descs.append(pltpu.async_copy(
                    ctx_hbm.at[xv[j]], cbuf.at[row, pl.ds(0, EMBED)], semc))
        for d in descs:
            d.wait()
        for cc in range(CHUNK // 16):
            res = jnp.zeros((L,), jnp.float32)
            for j in range(16):
                row = cc * 16 + j
                s = jnp.zeros((L,), jnp.float32)
                for k in range(NVREG):
                    s = s + (tbuf[row, pl.ds(k * 16, 16)]
                             * cbuf[row, pl.ds(k * 16, 16)])
                tot = jnp.sum(s)
                res = jnp.where(lane == j, tot, res)
            outv[pl.ds(cbase + cc * 16, 16)] = res
        return carry

    lax.fori_loop(0, NCHUNK, chunk_step, 0)
    pltpu.sync_copy(outv, out_hbm.at[pl.ds(base, BPW)])


@jax.jit
def _sc_call(x, t, target_rm, context_rm):
    mesh = plsc.VectorSubcoreMesh(core_axis_name="c", subcore_axis_name="s")
    return pl.kernel(
        _sc_body,
        out_type=jax.ShapeDtypeStruct((BATCH,), jnp.float32),
        mesh=mesh,
        compiler_params=pltpu.CompilerParams(
            needs_layout_passes=False,
        ),
        scratch_types=[
            pltpu.VMEM((BPW,), jnp.int32),
            pltpu.VMEM((BPW,), jnp.int32),
            pltpu.VMEM((CHUNK, 2 * EMBED), jnp.float32),
            pltpu.VMEM((CHUNK, 2 * EMBED), jnp.float32),
            pltpu.VMEM((BPW,), jnp.float32),
            pltpu.SemaphoreType.DMA,
            pltpu.SemaphoreType.DMA,
        ],
    )(x, t, target_rm, context_rm)


def kernel(x, t, target_table, context_table):
    target_rm, context_rm = _tc_transpose(target_table.T, context_table.T)
    return _sc_call(x, t, target_rm, context_rm)


# speedup vs baseline: 1.3038x; 1.3038x over previous
import jax
import jax.numpy as jnp
from jax import lax
from jax.experimental import pallas as pl
from jax.experimental.pallas import tpu as pltpu
from jax.experimental.pallas import tpu_sc as plsc

VOCAB = 1000000
EMBED = 64
BATCH = 16384

_info = plsc.get_sparse_core_info()
NC, NS, L = _info.num_cores, _info.num_subcores, _info.num_lanes
NW = NC * NS
BPW = BATCH // NW
CHUNK = 32
NCHUNK = BPW // CHUNK
NVREG = EMBED // 16


def _sc_body(x_hbm, t_hbm, tgt_hbm, ctx_hbm, out_hbm,
             xidx, tidx, tbuf, cbuf, outv, semt, semc):
    wid = lax.axis_index("s") * NC + lax.axis_index("c")
    base = wid * BPW

    pltpu.sync_copy(x_hbm.at[pl.ds(base, BPW)], xidx)
    pltpu.sync_copy(t_hbm.at[pl.ds(base, BPW)], tidx)

    lane = lax.iota(jnp.int32, L)

    def chunk_step(p, carry):
        cbase = p * CHUNK
        descs = []
        for cc in range(CHUNK // 16):
            tv = tidx[pl.ds(cbase + cc * 16, 16)]
            xv = xidx[pl.ds(cbase + cc * 16, 16)]
            for j in range(16):
                row = cc * 16 + j
                descs.append(pltpu.async_copy(
                    tgt_hbm.at[tv[j] >> 1], tbuf.at[row], semt))
                descs.append(pltpu.async_copy(
                    ctx_hbm.at[xv[j] >> 1], cbuf.at[row], semc))
        for d in descs:
            d.wait()
        for cc in range(CHUNK // 16):
            tv = tidx[pl.ds(cbase + cc * 16, 16)]
            xv = xidx[pl.ds(cbase + cc * 16, 16)]
            res = jnp.zeros((L,), jnp.float32)
            for j in range(16):
                row = cc * 16 + j
                todd = (tv[j] & 1) == 1
                xodd = (xv[j] & 1) == 1
                s = jnp.zeros((L,), jnp.float32)
                for k in range(NVREG):
                    tk = jnp.where(todd, tbuf[row, pl.ds(64 + k * 16, 16)],
                                   tbuf[row, pl.ds(k * 16, 16)])
                    ck = jnp.where(xodd, cbuf[row, pl.ds(64 + k * 16, 16)],
                                   cbuf[row, pl.ds(k * 16, 16)])
                    s = s + tk * ck
                tot = jnp.sum(s)
                res = jnp.where(lane == j, tot, res)
            outv[pl.ds(cbase + cc * 16, 16)] = res
        return carry

    lax.fori_loop(0, NCHUNK, chunk_step, 0)
    pltpu.sync_copy(outv, out_hbm.at[pl.ds(base, BPW)])


@jax.jit
def _sc_call(x, t, target_table, context_table):
    mesh = plsc.VectorSubcoreMesh(core_axis_name="c", subcore_axis_name="s")
    return pl.kernel(
        _sc_body,
        out_type=jax.ShapeDtypeStruct((BATCH,), jnp.float32),
        mesh=mesh,
        compiler_params=pltpu.CompilerParams(
            needs_layout_passes=False,
        ),
        scratch_types=[
            pltpu.VMEM((BPW,), jnp.int32),
            pltpu.VMEM((BPW,), jnp.int32),
            pltpu.VMEM((CHUNK, 2 * EMBED), jnp.float32),
            pltpu.VMEM((CHUNK, 2 * EMBED), jnp.float32),
            pltpu.VMEM((BPW,), jnp.float32),
            pltpu.SemaphoreType.DMA,
            pltpu.SemaphoreType.DMA,
        ],
    )(x, t, target_table, context_table)


def kernel(x, t, target_table, context_table):
    tgt = jnp.reshape(target_table, (VOCAB // 2, 2 * EMBED))
    ctx = jnp.reshape(context_table, (VOCAB // 2, 2 * EMBED))
    return _sc_call(x, t, tgt, ctx)


# speedup vs baseline: 2.0660x; 1.5846x over previous
import jax
import jax.numpy as jnp
from jax import lax
from jax.experimental import pallas as pl
from jax.experimental.pallas import tpu as pltpu
from jax.experimental.pallas import tpu_sc as plsc

VOCAB = 1000000
EMBED = 64
BATCH = 16384

_info = plsc.get_sparse_core_info()
NC, NS, L = _info.num_cores, _info.num_subcores, _info.num_lanes
NW = NC * NS
BPW = BATCH // NW
CHUNK = 32
NCHUNK = BPW // CHUNK
NVREG = EMBED // 16


def _sc_body(x_hbm, t_hbm, tgt_hbm, ctx_hbm, out_hbm,
             xidx, tidx, tbuf, cbuf, outv, semt, semc):
    wid = lax.axis_index("s") * NC + lax.axis_index("c")
    base = wid * BPW

    pltpu.sync_copy(x_hbm.at[pl.ds(base, BPW)], xidx)
    pltpu.sync_copy(t_hbm.at[pl.ds(base, BPW)], tidx)

    lane = lax.iota(jnp.int32, L)

    def chunk_step(p, carry):
        cbase = p * CHUNK
        descs = []
        for cc in range(CHUNK // 16):
            tv = tidx[pl.ds(cbase + cc * 16, 16)]
            xv = xidx[pl.ds(cbase + cc * 16, 16)]
            for j in range(16):
                row = cc * 16 + j
                descs.append(pltpu.async_copy(
                    tgt_hbm.at[tv[j]], tbuf.at[row, pl.ds(0, EMBED)], semt))
                descs.append(pltpu.async_copy(
                    ctx_hbm.at[xv[j]], cbuf.at[row, pl.ds(0, EMBED)], semc))
        for d in descs:
            d.wait()
        for cc in range(CHUNK // 16):
            res = jnp.zeros((L,), jnp.float32)
            for j in range(16):
                row = cc * 16 + j
                s = jnp.zeros((L,), jnp.float32)
                for k in range(NVREG):
                    s = s + (tbuf[row, pl.ds(k * 16, 16)]
                             * cbuf[row, pl.ds(k * 16, 16)])
                tot = jnp.sum(s)
                res = jnp.where(lane == j, tot, res)
            outv[pl.ds(cbase + cc * 16, 16)] = res
        return carry

    lax.fori_loop(0, NCHUNK, chunk_step, 0)
    pltpu.sync_copy(outv, out_hbm.at[pl.ds(base, BPW)])


@jax.jit
def _sc_call(x, t, target_table, context_table):
    mesh = plsc.VectorSubcoreMesh(core_axis_name="c", subcore_axis_name="s")
    return pl.kernel(
        _sc_body,
        out_type=jax.ShapeDtypeStruct((BATCH,), jnp.float32),
        mesh=mesh,
        compiler_params=pltpu.CompilerParams(
            needs_layout_passes=False,
        ),
        scratch_types=[
            pltpu.VMEM((BPW,), jnp.int32),
            pltpu.VMEM((BPW,), jnp.int32),
            pltpu.VMEM((CHUNK, 2 * EMBED), jnp.float32),
            pltpu.VMEM((CHUNK, 2 * EMBED), jnp.float32),
            pltpu.VMEM((BPW,), jnp.float32),
            pltpu.SemaphoreType.DMA,
            pltpu.SemaphoreType.DMA,
        ],
    )(x, t, target_table, context_table)


def kernel(x, t, target_table, context_table):
    return _sc_call(x, t, target_table, context_table)


# speedup vs baseline: 3.1359x; 1.5178x over previous
import jax
import jax.numpy as jnp
from jax import lax
from jax.experimental import pallas as pl
from jax.experimental.pallas import tpu as pltpu
from jax.experimental.pallas import tpu_sc as plsc

VOCAB = 1000000
EMBED = 64
BATCH = 16384

_info = plsc.get_sparse_core_info()
NC, NS, L = _info.num_cores, _info.num_subcores, _info.num_lanes
NW = NC * NS
BPW = BATCH // NW
CHUNK = 32
NCHUNK = BPW // CHUNK
NVREG = EMBED // 16


def _sc_body(x_hbm, t_hbm, tgt_hbm, ctx_hbm, out_hbm,
             xidx, tidx, tbuf, cbuf, outv, semt, semc):
    wid = lax.axis_index("s") * NC + lax.axis_index("c")
    base = wid * BPW

    pltpu.sync_copy(x_hbm.at[pl.ds(base, BPW)], xidx)
    pltpu.sync_copy(t_hbm.at[pl.ds(base, BPW)], tidx)

    lane = lax.iota(jnp.int32, L)

    def chunk_step(p, carry):
        cbase = p * CHUNK
        descs = []
        for cc in range(CHUNK // 16):
            tv = tidx[pl.ds(cbase + cc * 16, 16)]
            xv = xidx[pl.ds(cbase + cc * 16, 16)]
            for j in range(16):
                row = cc * 16 + j
                descs.append(pltpu.async_copy(
                    tgt_hbm.at[tv[j]], tbuf.at[row, pl.ds(0, EMBED)], semt))
                descs.append(pltpu.async_copy(
                    ctx_hbm.at[xv[j]], cbuf.at[row, pl.ds(0, EMBED)], semc))
        for d in descs:
            d.wait()
        for cc in range(CHUNK // 16):
            res = jnp.zeros((L,), jnp.float32)
            for j in range(16):
                row = cc * 16 + j
                s = jnp.zeros((L,), jnp.float32)
                for k in range(NVREG):
                    s = s + (tbuf[row, pl.ds(k * 16, 16)]
                             * cbuf[row, pl.ds(k * 16, 16)])
                tot = jnp.sum(s)
                res = jnp.where(lane == j, tot, res)
            outv[pl.ds(cbase + cc * 16, 16)] = res
        return carry

    lax.fori_loop(0, NCHUNK, chunk_step, 0)
    pltpu.sync_copy(outv, out_hbm.at[pl.ds(base, BPW)])


@jax.jit
def _sc_call(x, t, target_table, context_table):
    mesh = plsc.VectorSubcoreMesh(core_axis_name="c", subcore_axis_name="s")
    return pl.kernel(
        _sc_body,
        out_type=jax.ShapeDtypeStruct((BATCH,), jnp.float32),
        mesh=mesh,
        compiler_params=pltpu.CompilerParams(
            needs_layout_passes=False,
        ),
        scratch_types=[
            pltpu.VMEM((BPW,), jnp.int32),
            pltpu.VMEM((BPW,), jnp.int32),
            pltpu.VMEM((CHUNK, 2 * EMBED), jnp.float32),
            pltpu.VMEM((CHUNK, 2 * EMBED), jnp.float32),
            pltpu.VMEM((BPW,), jnp.float32),
            pltpu.SemaphoreType.DMA,
            pltpu.SemaphoreType.DMA,
        ],
    )(x, t, target_table, context_table)


def kernel(x, t, target_table, context_table):
    tt, cc = jax.lax.optimization_barrier(
        (target_table.T, context_table.T))
    return _sc_call(x, t, tt.T, cc.T)
